# baseline (speedup 1.0000x reference)
import functools

import jax
import jax.numpy as jnp
from jax.experimental import pallas as pl
from jax.experimental.pallas import tpu as pltpu

_EPS = 1e-6
_BR = 8192
_NUM_COL_CHUNKS = 8
_ST_ROWS = 512


def _bn_body(x_ref, o_ref, acc_ref, res_ref, *, br):
    p = pl.program_id(1)
    r = pl.program_id(2)

    @pl.when((p == 0) & (r == 0))
    def _():
        acc_ref[...] = jnp.zeros_like(acc_ref)

    @pl.when(p == 0)
    def _():
        base = r * br

        def step(i, carry):
            xc = x_ref[pl.ds(i * _ST_ROWS, _ST_ROWS), :]
            acc_ref[...] += jnp.sum(xc * xc, axis=0, keepdims=True)
            res_ref[pl.ds(base + i * _ST_ROWS, _ST_ROWS), :] = (
                xc.astype(jnp.bfloat16))
            return carry

        jax.lax.fori_loop(0, br // _ST_ROWS, step, 0)

    @pl.when(p == 1)
    def _():
        inv = jax.lax.rsqrt(acc_ref[...] + _EPS)
        xb = res_ref[pl.ds(r * br, br), :].astype(jnp.float32)
        o_ref[...] = xb * inv


def kernel(x):
    n, d = x.shape
    bc = d // _NUM_COL_CHUNKS
    br = min(_BR, n)
    num_row_blocks = n // br
    last = num_row_blocks - 1

    body = functools.partial(_bn_body, br=br)
    return pl.pallas_call(
        body,
        out_shape=jax.ShapeDtypeStruct((n, d), x.dtype),
        grid=(_NUM_COL_CHUNKS, 2, num_row_blocks),
        in_specs=[pl.BlockSpec(
            (br, bc), lambda c, p, r: (jnp.where(p == 0, r, last), c))],
        out_specs=pl.BlockSpec((br, bc), lambda c, p, r: (r * p, c)),
        scratch_shapes=[
            pltpu.VMEM((1, bc), jnp.float32),
            pltpu.VMEM((n, bc), jnp.bfloat16),
        ],
        compiler_params=pltpu.CompilerParams(
            dimension_semantics=("parallel", "arbitrary", "arbitrary"),
            vmem_limit_bytes=56 * 1024 * 1024,
        ),
        name="bn_colnorm_slabres",
    )(x)


# speedup vs baseline: 1.0772x; 1.0772x over previous
import functools

import jax
import jax.numpy as jnp
from jax.experimental import pallas as pl
from jax.experimental.pallas import tpu as pltpu

_EPS = 1e-6
_BR = 4096
_NUM_COL_CHUNKS = 8
_ST_ROWS = 512


def _bn_body(x_ref, o_ref, acc_ref, res_ref, *, br, num_chunks):
    t = pl.program_id(0)
    r = pl.program_id(1)
    fill = t % 2
    drain = (t + 1) % 2

    @pl.when(t < num_chunks)
    def _():
        @pl.when(r == 0)
        def _():
            acc_ref[fill] = jnp.zeros_like(acc_ref[fill])

        base = r * br

        def step(i, carry):
            xc = x_ref[pl.ds(i * _ST_ROWS, _ST_ROWS), :]
            acc_ref[fill] += jnp.sum(xc * xc, axis=0, keepdims=True)
            res_ref[fill, pl.ds(base + i * _ST_ROWS, _ST_ROWS), :] = (
                xc.astype(jnp.bfloat16))
            return carry

        jax.lax.fori_loop(0, br // _ST_ROWS, step, 0)

    @pl.when(t >= 1)
    def _():
        inv = jax.lax.rsqrt(acc_ref[drain] + _EPS)
        xb = res_ref[drain, pl.ds(r * br, br), :].astype(jnp.float32)
        o_ref[...] = xb * inv


def kernel(x):
    n, d = x.shape
    bc = d // _NUM_COL_CHUNKS
    br = min(_BR, n)
    num_row_blocks = n // br
    last_r = num_row_blocks - 1
    last_c = _NUM_COL_CHUNKS - 1

    def in_map(t, r):
        return (jnp.where(t < _NUM_COL_CHUNKS, r, last_r),
                jnp.minimum(t, last_c))

    def out_map(t, r):
        return (jnp.where(t >= 1, r, 0), jnp.maximum(t - 1, 0))

    body = functools.partial(_bn_body, br=br, num_chunks=_NUM_COL_CHUNKS)
    return pl.pallas_call(
        body,
        out_shape=jax.ShapeDtypeStruct((n, d), x.dtype),
        grid=(_NUM_COL_CHUNKS + 1, num_row_blocks),
        in_specs=[pl.BlockSpec((br, bc), in_map)],
        out_specs=pl.BlockSpec((br, bc), out_map),
        scratch_shapes=[
            pltpu.VMEM((2, 1, bc), jnp.float32),
            pltpu.VMEM((2, n, bc), jnp.bfloat16),
        ],
        compiler_params=pltpu.CompilerParams(
            dimension_semantics=("arbitrary", "arbitrary"),
            vmem_limit_bytes=56 * 1024 * 1024,
        ),
        name="bn_colnorm_pipe",
    )(x)


# speedup vs baseline: 1.1129x; 1.0332x over previous
import functools

import jax
import jax.numpy as jnp
from jax.experimental import pallas as pl
from jax.experimental.pallas import tpu as pltpu

_EPS = 1e-6
_BR = 16384
_NUM_COL_CHUNKS = 16
_ST_ROWS = 1024


def _bn_body(x_ref, o_ref, acc_ref, res_ref, *, br, num_chunks):
    t = pl.program_id(0)
    r = pl.program_id(1)
    fill = t % 2
    drain = (t + 1) % 2

    @pl.when(t < num_chunks)
    def _():
        @pl.when(r == 0)
        def _():
            acc_ref[fill] = jnp.zeros_like(acc_ref[fill])

        base = r * br

        def step(i, carry):
            xc = x_ref[pl.ds(i * _ST_ROWS, _ST_ROWS), :]
            acc_ref[fill] += jnp.sum(xc * xc, axis=0, keepdims=True)
            res_ref[fill, pl.ds(base + i * _ST_ROWS, _ST_ROWS), :] = (
                xc.astype(jnp.bfloat16))
            return carry

        jax.lax.fori_loop(0, br // _ST_ROWS, step, 0)

    @pl.when(t >= 1)
    def _():
        inv = jax.lax.rsqrt(acc_ref[drain] + _EPS)
        xb = res_ref[drain, pl.ds(r * br, br), :].astype(jnp.float32)
        o_ref[...] = xb * inv


def kernel(x):
    n, d = x.shape
    bc = d // _NUM_COL_CHUNKS
    br = min(_BR, n)
    num_row_blocks = n // br
    last_r = num_row_blocks - 1
    last_c = _NUM_COL_CHUNKS - 1

    def in_map(t, r):
        return (jnp.where(t < _NUM_COL_CHUNKS, r, last_r),
                jnp.minimum(t, last_c))

    def out_map(t, r):
        return (jnp.where(t >= 1, r, 0), jnp.maximum(t - 1, 0))

    body = functools.partial(_bn_body, br=br, num_chunks=_NUM_COL_CHUNKS)
    return pl.pallas_call(
        body,
        out_shape=jax.ShapeDtypeStruct((n, d), x.dtype),
        grid=(_NUM_COL_CHUNKS + 1, num_row_blocks),
        in_specs=[pl.BlockSpec((br, bc), in_map)],
        out_specs=pl.BlockSpec((br, bc), out_map),
        scratch_shapes=[
            pltpu.VMEM((2, 1, bc), jnp.float32),
            pltpu.VMEM((2, n, bc), jnp.bfloat16),
        ],
        compiler_params=pltpu.CompilerParams(
            dimension_semantics=("arbitrary", "arbitrary"),
            vmem_limit_bytes=56 * 1024 * 1024,
        ),
        name="bn_colnorm_pipe",
    )(x)
